# baseline (speedup 1.0000x reference)
import functools

import jax
import jax.numpy as jnp
from jax import lax
from jax.experimental import pallas as pl
from jax.experimental.pallas import tpu as pltpu
from jax.experimental.pallas import tpu_sc as plsc

NUM_CLASSES = 1000
B = 4096
S = 26
TOTAL_WORDS = S * NUM_CLASSES * B
N_WORKERS = 32
PAIRS_PER_WORKER = (S * B) // N_WORKERS
CHUNKS_PER_WORKER = PAIRS_PER_WORKER // 128


def _zero_body(out_ref):
    out_ref[...] = jnp.zeros_like(out_ref)


def _make_zeros():
    return pl.pallas_call(
        _zero_body,
        grid=(S,),
        out_specs=pl.BlockSpec((NUM_CLASSES, B), lambda s: (s, 0)),
        out_shape=jax.ShapeDtypeStruct((S * NUM_CLASSES, B), jnp.float32),
    )()


_sc_mesh = plsc.VectorSubcoreMesh(core_axis_name="c", subcore_axis_name="s")


@functools.partial(
    pl.kernel,
    mesh=_sc_mesh,
    scratch_types=[
        pltpu.VMEM((PAIRS_PER_WORKER,), jnp.int32),
        pltpu.VMEM((CHUNKS_PER_WORKER, 128), jnp.int32),
        pltpu.VMEM((CHUNKS_PER_WORKER, 128), jnp.float32),
        pltpu.SemaphoreType.DMA,
        pltpu.SemaphoreType.DMA,
    ],
)
def _sc_scatter(xt_hbm, out_hbm, xbuf, idxbuf, valbuf, lsem, ssem):
    wid = lax.axis_index("s") * 2 + lax.axis_index("c")
    q0 = wid * PAIRS_PER_WORKER
    pltpu.async_copy(xt_hbm.at[pl.ds(q0, PAIRS_PER_WORKER)], xbuf, lsem).wait()

    def fill_chunk(j, _):
        qc = q0 + j * 128
        for i in range(8):
            lane = i * 16 + lax.iota(jnp.int32, 16)
            q = qc + lane
            xv = xbuf[pl.ds(j * 128 + i * 16, 16)]
            pos = (q >> 12) * (NUM_CLASSES * B) + xv * B + (q & (B - 1))
            idxbuf[j, pl.ds(i * 16, 16)] = pos
            valbuf[j, pl.ds(i * 16, 16)] = jnp.full((16,), 1.0, jnp.float32)
        return 0

    lax.fori_loop(0, CHUNKS_PER_WORKER, fill_chunk, 0)

    def fire(j, _):
        pltpu.async_copy(valbuf.at[j], out_hbm.at[idxbuf.at[j]], ssem)
        return 0

    lax.fori_loop(0, CHUNKS_PER_WORKER, fire, 0)

    def drain(j, _):
        pltpu.make_async_copy(valbuf.at[j], out_hbm.at[idxbuf.at[j]], ssem).wait()
        return 0

    lax.fori_loop(0, CHUNKS_PER_WORKER, drain, 0)


def kernel(x):
    xt_flat = x.astype(jnp.int32).T.reshape(-1)
    buf = jax.new_ref(_make_zeros().reshape(-1))
    _sc_scatter(xt_flat, buf)
    out_flat = jax.freeze(buf)
    return out_flat.reshape(S, NUM_CLASSES, B).transpose(2, 0, 1)


# speedup vs baseline: 1.4346x; 1.4346x over previous
import functools

import jax
import jax.numpy as jnp
from jax import lax
from jax.experimental import pallas as pl
from jax.experimental.pallas import tpu as pltpu
from jax.experimental.pallas import tpu_sc as plsc

NUM_CLASSES = 1000
B = 4096
S = 26
TOTAL_WORDS = S * NUM_CLASSES * B
N_WORKERS = 32
PAIRS_PER_WORKER = (S * B) // N_WORKERS
CHUNKS_PER_WORKER = PAIRS_PER_WORKER // 128


def _zero_body(out_ref):
    out_ref[...] = jnp.zeros_like(out_ref)


def _make_zeros():
    return pl.pallas_call(
        _zero_body,
        grid=(S,),
        out_specs=pl.BlockSpec((NUM_CLASSES, B), lambda s: (s, 0)),
        out_shape=jax.ShapeDtypeStruct((S * NUM_CLASSES, B), jnp.float32),
    )()


_sc_mesh = plsc.VectorSubcoreMesh(core_axis_name="c", subcore_axis_name="s")


@functools.partial(
    pl.kernel,
    mesh=_sc_mesh,
    scratch_types=[
        pltpu.VMEM((PAIRS_PER_WORKER,), jnp.int32),
        pltpu.VMEM((PAIRS_PER_WORKER,), jnp.int32),
        pltpu.VMEM((PAIRS_PER_WORKER,), jnp.float32),
        pltpu.SemaphoreType.DMA,
        pltpu.SemaphoreType.DMA,
    ],
)
def _sc_scatter(xt_hbm, out_hbm, xbuf, idxbuf, valbuf, lsem, ssem):
    wid = lax.axis_index("s") * 2 + lax.axis_index("c")
    q0 = wid * PAIRS_PER_WORKER
    pltpu.async_copy(xt_hbm.at[pl.ds(q0, PAIRS_PER_WORKER)], xbuf, lsem).wait()

    def fill_chunk(j, _):
        qc = q0 + j * 128
        for i in range(8):
            lane = i * 16 + lax.iota(jnp.int32, 16)
            q = qc + lane
            xv = xbuf[pl.ds(j * 128 + i * 16, 16)]
            pos = (q >> 12) * (NUM_CLASSES * B) + xv * B + (q & (B - 1))
            idxbuf[pl.ds(j * 128 + i * 16, 16)] = pos
            valbuf[pl.ds(j * 128 + i * 16, 16)] = jnp.full((16,), 1.0, jnp.float32)
        return 0

    lax.fori_loop(0, CHUNKS_PER_WORKER, fill_chunk, 0)

    pltpu.async_copy(valbuf, out_hbm.at[idxbuf], ssem).wait()


def kernel(x):
    xt_flat = x.astype(jnp.int32).T.reshape(-1)
    buf = jax.new_ref(jnp.zeros((TOTAL_WORDS,), jnp.float32))
    _sc_scatter(xt_flat, buf)
    out_flat = jax.freeze(buf)
    return out_flat.reshape(S, NUM_CLASSES, B).transpose(2, 0, 1)


# speedup vs baseline: 7.3840x; 5.1472x over previous
import jax
import jax.numpy as jnp
from jax.experimental import pallas as pl

NUM_CLASSES = 1000


def _onehot_body(xt_ref, out_ref):
    xt = xt_ref[...]
    classes = jax.lax.broadcasted_iota(
        jnp.int32, (1, NUM_CLASSES, xt.shape[2]), 1
    )
    out_ref[...] = (xt == classes).astype(jnp.float32)


def kernel(x):
    B, S = x.shape
    xt = x.astype(jnp.int32).T
    xt = xt.reshape(S, 1, B)
    out = pl.pallas_call(
        _onehot_body,
        grid=(S,),
        in_specs=[pl.BlockSpec((1, 1, B), lambda s: (s, 0, 0))],
        out_specs=pl.BlockSpec((1, NUM_CLASSES, B), lambda s: (s, 0, 0)),
        out_shape=jax.ShapeDtypeStruct((S, NUM_CLASSES, B), jnp.float32),
    )(xt)
    return out.transpose(2, 0, 1)
